# baseline (speedup 1.0000x reference)
import functools

import jax
import jax.numpy as jnp
from jax import lax
from jax.experimental import pallas as pl
from jax.experimental.pallas import tpu as pltpu
from jax.experimental.pallas import tpu_sc as plsc

VOCAB = 100000
BATCH = 4096
SEQ = 200
EMB = 64
HALF = SEQ // 2
NC = 2
NS = 16
NW = NC * NS
BPW = BATCH // NW
NEG_INF = float("-inf")
NBUF = 4


def _pool_body(table_hbm, x2_hbm, out_hbm, idx_v, buf_v, pooled_v, *sems):
    wid = lax.axis_index("s") * NC + lax.axis_index("c")
    base = wid * BPW
    pltpu.sync_copy(x2_hbm.at[pl.ds(base, BPW)], idx_v)

    splits = ((0, 104), (104, 96))

    def fire(row, st):
        for off, ln in splits:
            pltpu.async_copy(
                table_hbm.at[idx_v.at[row, pl.ds(off, ln)]],
                buf_v.at[st, pl.ds(off, ln)],
                sems[st],
            )

    def drain(st):
        for off, ln in splits:
            pltpu.make_async_copy(
                table_hbm.at[idx_v.at[0, pl.ds(off, ln)]],
                buf_v.at[st, pl.ds(off, ln)],
                sems[st],
            ).wait()

    def compute(row, st):
        def max_step(r, accs):
            new = list(accs)
            for u in range(2):
                for c in range(4):
                    v = buf_v[st, 2 * r + u, pl.ds(c * 16, 16)]
                    k = u * 4 + c
                    new[k] = jnp.maximum(new[k], v)
            return tuple(new)

        init = tuple(jnp.full((16,), NEG_INF, jnp.float32) for _ in range(8))
        accs = lax.fori_loop(0, SEQ // 2, max_step, init)
        for c in range(4):
            pooled_v[row, pl.ds(c * 16, 16)] = jnp.maximum(accs[c], accs[4 + c])

    for st in range(NBUF - 1):
        fire(st, st)

    def ring_step(j, _):
        row0 = NBUF * j
        for st in range(NBUF):
            row = row0 + st
            ahead = row + NBUF - 1

            @pl.when(ahead < BPW)
            def _():
                fire(ahead, (st + NBUF - 1) % NBUF)

            drain(st)
            compute(row, st)
        return ()

    lax.fori_loop(0, BPW // NBUF, ring_step, ())
    pltpu.sync_copy(pooled_v, out_hbm.at[pl.ds(base, BPW)])


def _sc_pool(x2, table):
    mesh = plsc.VectorSubcoreMesh(
        core_axis_name="c", subcore_axis_name="s", num_cores=NC, num_subcores=NS
    )
    fn = pl.kernel(
        _pool_body,
        out_type=jax.ShapeDtypeStruct((BATCH, EMB), jnp.float32),
        mesh=mesh,
        scratch_types=[
            pltpu.VMEM((BPW, SEQ), jnp.int32),
            pltpu.VMEM((NBUF, SEQ, EMB), jnp.float32),
            pltpu.VMEM((BPW, EMB), jnp.float32),
        ]
        + [pltpu.SemaphoreType.DMA] * NBUF,
        compiler_params=pltpu.CompilerParams(use_tc_tiling_on_sc=False),
    )
    return fn(table, x2)


NTC = 782
VPAD = NTC * 128
NSTG = 3


def _fmt_body(tt_hbm, p_hbm, in_v, out_v, *sems):
    wid = lax.axis_index("s") * NC + lax.axis_index("c")
    sin = sems[:NSTG]
    sout = sems[NSTG:]
    iota = lax.iota(jnp.int32, 16)
    kmax = (NTC + NW - 1) // NW

    def fire_in(k, st):
        tc = wid + NW * k
        off = pl.multiple_of(tc * 128, 128)
        pltpu.async_copy(tt_hbm.at[:, pl.ds(off, 128)], in_v.at[st], sin[st])

    def wait_in(st):
        pltpu.make_async_copy(
            tt_hbm.at[:, pl.ds(0, 128)], in_v.at[st], sin[st]
        ).wait()

    def fire_out(k, st):
        tc = wid + NW * k
        off = pl.multiple_of(tc * 64, 64)
        pltpu.async_copy(out_v.at[st], p_hbm.at[pl.ds(off, 64)], sout[st])

    def wait_out(st):
        pltpu.make_async_copy(
            out_v.at[st], p_hbm.at[pl.ds(0, 64)], sout[st]
        ).wait()

    def transpose_block(st):
        def eb_step(eb, _):
            e0 = eb * 16

            def g_step(g, __):
                lv = iota + g * 16
                prow = lax.shift_right_logical(lv, 1)
                pbase = lax.shift_left(jnp.bitwise_and(lv, 1), 6)
                for d in range(16):
                    rv = jnp.bitwise_and(iota + d, 15) + e0
                    val = plsc.load_gather(in_v.at[st], [rv, lv])
                    plsc.store_scatter(out_v.at[st], [prow, pbase + rv], val)
                return ()

            lax.fori_loop(0, 8, g_step, ())
            return ()

        lax.fori_loop(0, 4, eb_step, ())

    for st in range(NSTG - 1):
        fire_in(st, st)
    for k in range(kmax):
        st = k % NSTG
        ok = (wid + NW * k) < NTC
        if k + NSTG - 1 < kmax:
            @pl.when((wid + NW * (k + NSTG - 1)) < NTC)
            def _():
                fire_in(k + NSTG - 1, (k + NSTG - 1) % NSTG)

        @pl.when(ok)
        def _():
            wait_in(st)

        if k >= NSTG:
            @pl.when(ok)
            def _():
                wait_out(st)

        @pl.when(ok)
        def _():
            transpose_block(st)
            fire_out(k, st)

    for k in range(kmax - NSTG, kmax):
        @pl.when((wid + NW * k) < NTC)
        def _():
            wait_out(k % NSTG)


def _sc_format(table):
    tt = table.T
    mesh = plsc.VectorSubcoreMesh(
        core_axis_name="c", subcore_axis_name="s", num_cores=NC, num_subcores=NS
    )
    fn = pl.kernel(
        _fmt_body,
        out_type=jax.ShapeDtypeStruct((VPAD // 2, 128), jnp.float32),
        mesh=mesh,
        scratch_types=[
            pltpu.VMEM((NSTG, EMB, 128), jnp.float32),
            pltpu.VMEM((NSTG, EMB, 128), jnp.float32),
        ]
        + [pltpu.SemaphoreType.DMA] * (2 * NSTG),
        compiler_params=pltpu.CompilerParams(
            use_tc_tiling_on_sc=True, needs_layout_passes=False
        ),
    )
    return fn(tt)


def _linear_body(p_ref, w_ref, b_ref, o_ref):
    o_ref[...] = (
        lax.dot_general(
            p_ref[...], w_ref[...], (((1,), (1,)), ((), ())),
            preferred_element_type=jnp.float32,
        )
        + b_ref[...]
    )


def _tc_linear(pooled, W, b):
    return pl.pallas_call(
        _linear_body,
        out_shape=jax.ShapeDtypeStruct((BATCH, 2), jnp.float32),
    )(pooled, W, b.reshape(1, 2))


@jax.jit
def kernel(x, table, W, b):
    packed = _sc_format(table)
    view = packed.reshape(VPAD, EMB)
    pooled = _sc_pool(x.astype(jnp.int32), view)
    return _tc_linear(pooled, W, b)


# speedup vs baseline: 1.2617x; 1.2617x over previous
import functools

import jax
import jax.numpy as jnp
from jax import lax
from jax.experimental import pallas as pl
from jax.experimental.pallas import tpu as pltpu
from jax.experimental.pallas import tpu_sc as plsc

VOCAB = 100000
BATCH = 4096
SEQ = 200
EMB = 64
HALF = SEQ // 2
NC = 2
NS = 16
NW = NC * NS
BPW = BATCH // NW
NEG_INF = float("-inf")
NBUF = 4


def _pool_body(table_hbm, x2_hbm, out_hbm, idx_v, buf_v, pooled_v, *sems):
    wid = lax.axis_index("s") * NC + lax.axis_index("c")
    base = wid * BPW
    pltpu.sync_copy(x2_hbm.at[pl.ds(base, BPW)], idx_v)

    splits = ((0, 104), (104, 96))

    def fire(row, st):
        for off, ln in splits:
            pltpu.async_copy(
                table_hbm.at[idx_v.at[row, pl.ds(off, ln)]],
                buf_v.at[st, pl.ds(off, ln)],
                sems[st],
            )

    def drain(st):
        for off, ln in splits:
            pltpu.make_async_copy(
                table_hbm.at[idx_v.at[0, pl.ds(off, ln)]],
                buf_v.at[st, pl.ds(off, ln)],
                sems[st],
            ).wait()

    def compute(row, st):
        def max_step(r, accs):
            new = list(accs)
            for u in range(2):
                for c in range(4):
                    v = buf_v[st, 2 * r + u, pl.ds(c * 16, 16)]
                    k = u * 4 + c
                    new[k] = jnp.maximum(new[k], v)
            return tuple(new)

        init = tuple(jnp.full((16,), NEG_INF, jnp.float32) for _ in range(8))
        accs = lax.fori_loop(0, SEQ // 2, max_step, init)
        for c in range(4):
            pooled_v[row, pl.ds(c * 16, 16)] = jnp.maximum(accs[c], accs[4 + c])

    for st in range(NBUF - 1):
        fire(st, st)

    def ring_step(j, _):
        row0 = NBUF * j
        for st in range(NBUF):
            row = row0 + st
            ahead = row + NBUF - 1

            @pl.when(ahead < BPW)
            def _():
                fire(ahead, (st + NBUF - 1) % NBUF)

            drain(st)
            compute(row, st)
        return ()

    lax.fori_loop(0, BPW // NBUF, ring_step, ())
    pltpu.sync_copy(pooled_v, out_hbm.at[pl.ds(base, BPW)])


def _sc_pool(x2, table):
    mesh = plsc.VectorSubcoreMesh(
        core_axis_name="c", subcore_axis_name="s", num_cores=NC, num_subcores=NS
    )
    fn = pl.kernel(
        _pool_body,
        out_type=jax.ShapeDtypeStruct((BATCH, EMB), jnp.float32),
        mesh=mesh,
        scratch_types=[
            pltpu.VMEM((BPW, SEQ), jnp.int32),
            pltpu.VMEM((NBUF, SEQ, EMB), jnp.float32),
            pltpu.VMEM((BPW, EMB), jnp.float32),
        ]
        + [pltpu.SemaphoreType.DMA] * NBUF,
        compiler_params=pltpu.CompilerParams(use_tc_tiling_on_sc=False),
    )
    return fn(table, x2)


NTC = 782
VPAD = NTC * 128
NSTG = 3


def _fmt_body(tt_hbm, p_hbm, in_v, out_v, *sems):
    wid = lax.axis_index("s") * NC + lax.axis_index("c")
    sin = sems[:NSTG]
    sout = sems[NSTG:]
    iota = lax.iota(jnp.int32, 16)
    kmax = (NTC + NW - 1) // NW

    def fire_in(k, st):
        tc = wid + NW * k
        off = pl.multiple_of(tc * 128, 128)
        pltpu.async_copy(tt_hbm.at[:, pl.ds(off, 128)], in_v.at[st], sin[st])

    def wait_in(st):
        pltpu.make_async_copy(
            tt_hbm.at[:, pl.ds(0, 128)], in_v.at[st], sin[st]
        ).wait()

    def fire_out(k, st):
        tc = wid + NW * k
        off = pl.multiple_of(tc * 64, 64)
        pltpu.async_copy(out_v.at[st], p_hbm.at[pl.ds(off, 64)], sout[st])

    def wait_out(st):
        pltpu.make_async_copy(
            out_v.at[st], p_hbm.at[pl.ds(0, 64)], sout[st]
        ).wait()

    rv16 = tuple(jnp.bitwise_and(iota + d, 15) for d in range(16))

    def transpose_block(st):
        def eb_step(eb, _):
            e0 = eb * 16

            def g_step(g, __):
                lv = iota + g * 16
                prow = lax.shift_right_logical(lv, 1)
                pbase = lax.shift_left(jnp.bitwise_and(lv, 1), 6) + e0
                rvs = [rv16[d] + e0 for d in range(16)]
                vals = [
                    plsc.load_gather(in_v.at[st], [rvs[d], lv])
                    for d in range(16)
                ]
                for d in range(16):
                    plsc.store_scatter(
                        out_v.at[st], [prow, pbase + rv16[d]], vals[d]
                    )
                return ()

            lax.fori_loop(0, 8, g_step, ())
            return ()

        lax.fori_loop(0, 4, eb_step, ())

    for st in range(NSTG - 1):
        fire_in(st, st)
    for k in range(kmax):
        st = k % NSTG
        ok = (wid + NW * k) < NTC
        if k + NSTG - 1 < kmax:
            @pl.when((wid + NW * (k + NSTG - 1)) < NTC)
            def _():
                fire_in(k + NSTG - 1, (k + NSTG - 1) % NSTG)

        @pl.when(ok)
        def _():
            wait_in(st)

        if k >= NSTG:
            @pl.when(ok)
            def _():
                wait_out(st)

        @pl.when(ok)
        def _():
            transpose_block(st)
            fire_out(k, st)

    for k in range(kmax - NSTG, kmax):
        @pl.when((wid + NW * k) < NTC)
        def _():
            wait_out(k % NSTG)


def _sc_format(table):
    tt = table.T
    mesh = plsc.VectorSubcoreMesh(
        core_axis_name="c", subcore_axis_name="s", num_cores=NC, num_subcores=NS
    )
    fn = pl.kernel(
        _fmt_body,
        out_type=jax.ShapeDtypeStruct((VPAD // 2, 128), jnp.float32),
        mesh=mesh,
        scratch_types=[
            pltpu.VMEM((NSTG, EMB, 128), jnp.float32),
            pltpu.VMEM((NSTG, EMB, 128), jnp.float32),
        ]
        + [pltpu.SemaphoreType.DMA] * (2 * NSTG),
        compiler_params=pltpu.CompilerParams(
            use_tc_tiling_on_sc=True, needs_layout_passes=False
        ),
    )
    return fn(tt)


def _linear_body(p_ref, w_ref, b_ref, o_ref):
    o_ref[...] = (
        lax.dot_general(
            p_ref[...], w_ref[...], (((1,), (1,)), ((), ())),
            preferred_element_type=jnp.float32,
        )
        + b_ref[...]
    )


def _tc_linear(pooled, W, b):
    return pl.pallas_call(
        _linear_body,
        out_shape=jax.ShapeDtypeStruct((BATCH, 2), jnp.float32),
    )(pooled, W, b.reshape(1, 2))


@jax.jit
def kernel(x, table, W, b):
    packed = _sc_format(table)
    view = packed.reshape(VPAD, EMB)
    pooled = _sc_pool(x.astype(jnp.int32), view)
    return _tc_linear(pooled, W, b)
